# baseline (speedup 1.0000x reference)
import functools
import math

import jax
import jax.numpy as jnp
import numpy as np
from jax import lax
from jax.experimental import pallas as pl
from jax.experimental.pallas import tpu as pltpu
from jax.experimental.pallas import tpu_sc as plsc


def _make_sinusoidal_pe(max_len, embed_dim):
    pe = np.zeros((max_len, embed_dim), dtype=np.float32)
    position = np.arange(0, max_len, dtype=np.float32)[:, None]
    div_term = np.exp(
        np.arange(0, embed_dim, 2, dtype=np.float32) * -(math.log(10000.0) / embed_dim)
    )
    pe[:, 0::2] = np.sin(position * div_term)
    pe[:, 1::2] = np.cos(position * div_term)
    return pe


_PE = _make_sinusoidal_pe(512, 64)

_RING = 2
_GRP = 5


def _sc_embed_planes(tableT, idxT, peT):
    nc, ns = 2, 16
    nw = nc * ns
    D, V = tableT.shape
    S, B = idxT.shape
    R, G = _RING, _GRP
    assert D % nw == 0 and B == 1024 and S % (R * G) == 0
    passes = D // nw
    ngroups = S // G

    mesh = plsc.VectorSubcoreMesh(
        core_axis_name="c", subcore_axis_name="s", num_cores=nc, num_subcores=ns
    )

    @functools.partial(
        pl.kernel,
        out_type=jax.ShapeDtypeStruct((S, 8, 8, 8, 128), jnp.float32),
        mesh=mesh,
        scratch_types=[
            pltpu.VMEM((S,), jnp.float32),
            pltpu.VMEM((V,), jnp.float32),
            pltpu.VMEM((R, G, B), jnp.int32),
            pltpu.VMEM((R, G, 8, 1, 128), jnp.float32),
        ]
        + [pltpu.SemaphoreType.DMA for _ in range(2 * R)],
        compiler_params=pltpu.CompilerParams(
            use_tc_tiling_on_sc=False, needs_layout_passes=False
        ),
    )
    def k(tableT_hbm, idxT_hbm, peT_hbm, out_hbm, pe_s, trow_v, ibuf, obuf, *sems):
        isem = sems[:R]
        osem = sems[R : 2 * R]
        wid = lax.axis_index("s") * nc + lax.axis_index("c")

        def idx_fetch(gi, q):
            return pltpu.make_async_copy(
                idxT_hbm.at[pl.ds(gi * G, G)], ibuf.at[q], isem[q]
            )

        for p in range(passes):
            d = wid + p * nw
            dh = d // 8
            dl = lax.rem(d, 8)
            pltpu.sync_copy(peT_hbm.at[d], pe_s)
            pltpu.sync_copy(tableT_hbm.at[d], trow_v)

            def writeback(gi, q, dh=dh, dl=dl):
                return pltpu.make_async_copy(
                    obuf.at[q],
                    out_hbm.at[pl.ds(gi * G, G), dh, :, pl.ds(dl, 1), :],
                    osem[q],
                )

            for q in range(R):
                idx_fetch(q, q).start()

            def step(g, carry, d=d, writeback=writeback):
                for q in range(R):
                    gi = R * g + q
                    idx_fetch(gi, q).wait()

                    @pl.when(gi >= R)
                    def _():
                        writeback(gi - R, q).wait()

                    for j in range(G):
                        s = gi * G + j
                        vpe = plsc.load_gather(
                            pe_s, [jnp.full((16,), s, jnp.int32)]
                        )
                        for r in range(8):
                            for jj in range(8):
                                idxv = ibuf[q, j, pl.ds(r * 128 + jj * 16, 16)]
                                vals = plsc.load_gather(trow_v, [idxv])
                                obuf[q, j, r, 0, pl.ds(jj * 16, 16)] = vals + vpe
                    writeback(gi, q).start()

                    @pl.when(gi + R < ngroups)
                    def _():
                        idx_fetch(gi + R, q).start()

                return carry

            lax.fori_loop(0, ngroups // R, step, 0)
            for q in range(R):
                writeback(ngroups - R + q, q).wait()

    return k(tableT, idxT, peT)


@jax.jit
def kernel(input, word_table):
    B, S = input.shape
    D = word_table.shape[1]
    peT = jnp.asarray(np.ascontiguousarray(_PE[:S, :D].T))
    o5 = _sc_embed_planes(word_table.T, input.T, peT)
    return o5.transpose(2, 4, 0, 1, 3).reshape(B, S, D)


# speedup vs baseline: 1.0139x; 1.0139x over previous
import functools
import math

import jax
import jax.numpy as jnp
import numpy as np
from jax import lax
from jax.experimental import pallas as pl
from jax.experimental.pallas import tpu as pltpu
from jax.experimental.pallas import tpu_sc as plsc


def _make_sinusoidal_pe(max_len, embed_dim):
    pe = np.zeros((max_len, embed_dim), dtype=np.float32)
    position = np.arange(0, max_len, dtype=np.float32)[:, None]
    div_term = np.exp(
        np.arange(0, embed_dim, 2, dtype=np.float32) * -(math.log(10000.0) / embed_dim)
    )
    pe[:, 0::2] = np.sin(position * div_term)
    pe[:, 1::2] = np.cos(position * div_term)
    return pe


_PE = _make_sinusoidal_pe(512, 64)

_RING = 2
_GRP = 4


def _sc_embed_planes(tableT, idxT, peT):
    nc, ns = 2, 16
    nw = nc * ns
    D, V = tableT.shape
    S, B = idxT.shape
    R, G = _RING, _GRP
    assert D % nw == 0 and B == 1024 and S % (R * G) == 0
    passes = D // nw
    ngroups = S // G

    mesh = plsc.VectorSubcoreMesh(
        core_axis_name="c", subcore_axis_name="s", num_cores=nc, num_subcores=ns
    )

    @functools.partial(
        pl.kernel,
        out_type=jax.ShapeDtypeStruct((S, 8, 8, 8, 128), jnp.float32),
        mesh=mesh,
        scratch_types=[
            pltpu.VMEM((S,), jnp.float32),
            pltpu.VMEM((V,), jnp.float32),
            pltpu.VMEM((R, G, B), jnp.int32),
            pltpu.VMEM((R, G, 8, 1, 128), jnp.float32),
            pltpu.VMEM_SHARED((S, B), jnp.int32),
        ]
        + [pltpu.SemaphoreType.DMA for _ in range(2 * R)],
        compiler_params=pltpu.CompilerParams(
            use_tc_tiling_on_sc=False, needs_layout_passes=False
        ),
    )
    def k(
        tableT_hbm, idxT_hbm, peT_hbm, out_hbm, pe_s, trow_v, ibuf, obuf, sidx, *sems
    ):
        isem = sems[:R]
        osem = sems[R : 2 * R]
        sid = lax.axis_index("s")
        wid = sid * nc + lax.axis_index("c")

        rows = S // 8
        @pl.when(sid < 8)
        def _():
            pltpu.sync_copy(
                idxT_hbm.at[pl.ds(sid * rows, rows)],
                sidx.at[pl.ds(sid * rows, rows)],
            )

        plsc.subcore_barrier()

        def idx_fetch(gi, q):
            return pltpu.make_async_copy(
                sidx.at[pl.ds(gi * G, G)], ibuf.at[q], isem[q]
            )

        for p in range(passes):
            d = wid + p * nw
            dh = d // 8
            dl = lax.rem(d, 8)
            pltpu.sync_copy(peT_hbm.at[d], pe_s)
            pltpu.sync_copy(tableT_hbm.at[d], trow_v)

            def writeback(gi, q, dh=dh, dl=dl):
                return pltpu.make_async_copy(
                    obuf.at[q],
                    out_hbm.at[pl.ds(gi * G, G), dh, :, pl.ds(dl, 1), :],
                    osem[q],
                )

            for q in range(R):
                idx_fetch(q, q).start()

            def step(g, carry, d=d, writeback=writeback):
                for q in range(R):
                    gi = R * g + q
                    idx_fetch(gi, q).wait()

                    @pl.when(gi >= R)
                    def _():
                        writeback(gi - R, q).wait()

                    for j in range(G):
                        s = gi * G + j
                        vpe = plsc.load_gather(
                            pe_s, [jnp.full((16,), s, jnp.int32)]
                        )
                        for r in range(8):
                            for jj in range(8):
                                idxv = ibuf[q, j, pl.ds(r * 128 + jj * 16, 16)]
                                vals = plsc.load_gather(trow_v, [idxv])
                                obuf[q, j, r, 0, pl.ds(jj * 16, 16)] = vals + vpe
                    writeback(gi, q).start()

                    @pl.when(gi + R < ngroups)
                    def _():
                        idx_fetch(gi + R, q).start()

                return carry

            lax.fori_loop(0, ngroups // R, step, 0)
            for q in range(R):
                writeback(ngroups - R + q, q).wait()

    return k(tableT, idxT, peT)


@jax.jit
def kernel(input, word_table):
    B, S = input.shape
    D = word_table.shape[1]
    peT = jnp.asarray(np.ascontiguousarray(_PE[:S, :D].T))
    o5 = _sc_embed_planes(word_table.T, input.T, peT)
    return o5.transpose(2, 4, 0, 1, 3).reshape(B, S, D)


# speedup vs baseline: 1.0854x; 1.0705x over previous
import functools
import math

import jax
import jax.numpy as jnp
import numpy as np
from jax import lax
from jax.experimental import pallas as pl
from jax.experimental.pallas import tpu as pltpu
from jax.experimental.pallas import tpu_sc as plsc


def _make_sinusoidal_pe(max_len, embed_dim):
    pe = np.zeros((max_len, embed_dim), dtype=np.float32)
    position = np.arange(0, max_len, dtype=np.float32)[:, None]
    div_term = np.exp(
        np.arange(0, embed_dim, 2, dtype=np.float32) * -(math.log(10000.0) / embed_dim)
    )
    pe[:, 0::2] = np.sin(position * div_term)
    pe[:, 1::2] = np.cos(position * div_term)
    return pe


_PE = _make_sinusoidal_pe(512, 64)

_RING = 2
_GRP = 4


def _sc_embed_planes(tableT, idxT, peT):
    nc, ns = 2, 16
    nw = nc * ns
    D, V = tableT.shape
    S = idxT.shape[0]
    B = idxT.shape[1] * idxT.shape[3]
    R, G = _RING, _GRP
    assert D % nw == 0 and B == 1024 and S % (R * G) == 0
    passes = D // nw
    ngroups = S // G

    mesh = plsc.VectorSubcoreMesh(
        core_axis_name="c", subcore_axis_name="s", num_cores=nc, num_subcores=ns
    )

    @functools.partial(
        pl.kernel,
        out_type=jax.ShapeDtypeStruct((S, 8, 8, 8, 128), jnp.float32),
        mesh=mesh,
        scratch_types=[
            pltpu.VMEM((S,), jnp.float32),
            pltpu.VMEM((R, G, 8, 1, 128), jnp.int32),
            pltpu.VMEM((R, G, 8, 1, 128), jnp.float32),
            pltpu.VMEM_SHARED((S, 8, 1, 128), jnp.int32),
            pltpu.VMEM_SHARED((16, V), jnp.float32),
        ]
        + [pltpu.SemaphoreType.DMA for _ in range(3 * R)],
        compiler_params=pltpu.CompilerParams(
            use_tc_tiling_on_sc=False, needs_layout_passes=False
        ),
    )
    def k(tableT_hbm, idxT_hbm, peT_hbm, out_hbm, pe_s, ibuf, obuf, sidx, srows, *sems):
        isem = sems[:R]
        osem = sems[R : 2 * R]
        gsem = sems[2 * R : 3 * R]
        sid = lax.axis_index("s")
        wid = sid * nc + lax.axis_index("c")

        rows = S // 8

        @pl.when(sid < 8)
        def _():
            pltpu.sync_copy(
                idxT_hbm.at[pl.ds(sid * rows, rows)],
                sidx.at[pl.ds(sid * rows, rows)],
            )

        def idx_fetch(gi, q):
            return pltpu.make_async_copy(
                sidx.at[pl.ds(gi * G, G)], ibuf.at[q], isem[q]
            )

        for p in range(passes):
            d = wid + p * nw
            dh = d // 8
            dl = lax.rem(d, 8)
            pltpu.sync_copy(peT_hbm.at[d], pe_s)
            pltpu.sync_copy(tableT_hbm.at[d], srows.at[sid])
            plsc.subcore_barrier()
            myrow = srows.at[sid]

            def writeback(gi, q, dh=dh, dl=dl):
                return pltpu.make_async_copy(
                    obuf.at[q],
                    out_hbm.at[pl.ds(gi * G, G), dh, :, pl.ds(dl, 1), :],
                    osem[q],
                )

            def gather_add(q, j, myrow=myrow):
                return [
                    pltpu.make_async_copy(
                        myrow.at[ibuf.at[q, j, r, 0]],
                        obuf.at[q, j, r, 0],
                        gsem[q],
                    )
                    for r in range(8)
                ]

            for q in range(R):
                idx_fetch(q, q).start()

            def step(g, carry, writeback=writeback, gather_add=gather_add):
                for q in range(R):
                    gi = R * g + q
                    idx_fetch(gi, q).wait()

                    @pl.when(gi >= R)
                    def _():
                        writeback(gi - R, q).wait()

                    for j in range(G):
                        s = gi * G + j
                        vpe = plsc.load_gather(
                            pe_s, [jnp.full((16,), s, jnp.int32)]
                        )
                        for r in range(8):
                            for jj in range(8):
                                obuf[q, j, r, 0, pl.ds(jj * 16, 16)] = vpe
                        for c in gather_add(q, j):
                            c.start(add=True)
                    for j in range(G):
                        for c in gather_add(q, j):
                            c.wait()
                    writeback(gi, q).start()

                    @pl.when(gi + R < ngroups)
                    def _():
                        idx_fetch(gi + R, q).start()

                return carry

            lax.fori_loop(0, ngroups // R, step, 0)
            for q in range(R):
                writeback(ngroups - R + q, q).wait()
            plsc.subcore_barrier()

    return k(tableT, idxT, peT)


@jax.jit
def kernel(input, word_table):
    B, S = input.shape
    D = word_table.shape[1]
    peT = jnp.asarray(np.ascontiguousarray(_PE[:S, :D].T))
    o5 = _sc_embed_planes(word_table.T, input.T.reshape(S, 8, 1, 128), peT)
    return o5.transpose(2, 4, 0, 1, 3).reshape(B, S, D)


# speedup vs baseline: 1.1409x; 1.0512x over previous
import functools
import math

import jax
import jax.numpy as jnp
import numpy as np
from jax import lax
from jax.experimental import pallas as pl
from jax.experimental.pallas import tpu as pltpu
from jax.experimental.pallas import tpu_sc as plsc


def _make_sinusoidal_pe(max_len, embed_dim):
    pe = np.zeros((max_len, embed_dim), dtype=np.float32)
    position = np.arange(0, max_len, dtype=np.float32)[:, None]
    div_term = np.exp(
        np.arange(0, embed_dim, 2, dtype=np.float32) * -(math.log(10000.0) / embed_dim)
    )
    pe[:, 0::2] = np.sin(position * div_term)
    pe[:, 1::2] = np.cos(position * div_term)
    return pe


_PE = _make_sinusoidal_pe(512, 64)

_RING = 2
_GRP = 4


def _sc_embed_planes(tableT, idxT, peT):
    nc, ns = 2, 16
    nw = nc * ns
    D, V = tableT.shape
    S = idxT.shape[0]
    B = idxT.shape[1] * idxT.shape[3]
    R, G = _RING, _GRP
    assert D % nw == 0 and B == 1024 and S % (R * G) == 0
    passes = D // nw
    ngroups = S // G

    mesh = plsc.VectorSubcoreMesh(
        core_axis_name="c", subcore_axis_name="s", num_cores=nc, num_subcores=ns
    )

    @functools.partial(
        pl.kernel,
        out_type=jax.ShapeDtypeStruct((S, 8, 8, 8, 128), jnp.float32),
        mesh=mesh,
        scratch_types=[
            pltpu.VMEM((S,), jnp.float32),
            pltpu.VMEM((R, G, 8, 1, 128), jnp.int32),
            pltpu.VMEM((R, G, 8, 1, 128), jnp.float32),
            pltpu.VMEM_SHARED((S, 8, 1, 128), jnp.int32),
            pltpu.VMEM_SHARED((16, V), jnp.float32),
        ]
        + [pltpu.SemaphoreType.DMA for _ in range(3 * R)],
        compiler_params=pltpu.CompilerParams(
            use_tc_tiling_on_sc=False, needs_layout_passes=False
        ),
    )
    def k(tableT_hbm, idxT_hbm, peT_hbm, out_hbm, pe_s, ibuf, obuf, sidx, srows, *sems):
        isem = sems[:R]
        osem = sems[R : 2 * R]
        gsem = sems[2 * R : 3 * R]
        sid = lax.axis_index("s")
        wid = sid * nc + lax.axis_index("c")

        rows = S // 8

        @pl.when(sid < 8)
        def _():
            pltpu.sync_copy(
                idxT_hbm.at[pl.ds(sid * rows, rows)],
                sidx.at[pl.ds(sid * rows, rows)],
            )

        def idx_fetch(gi, q):
            return pltpu.make_async_copy(
                sidx.at[pl.ds(gi * G, G)], ibuf.at[q], isem[q]
            )

        for p in range(passes):
            d = wid + p * nw
            dh = d // 8
            dl = lax.rem(d, 8)
            pltpu.sync_copy(peT_hbm.at[d], pe_s)
            pltpu.sync_copy(tableT_hbm.at[d], srows.at[sid])
            plsc.subcore_barrier()
            myrow = srows.at[sid]

            def writeback(gi, q, dh=dh, dl=dl):
                return pltpu.make_async_copy(
                    obuf.at[q],
                    out_hbm.at[pl.ds(gi * G, G), dh, :, pl.ds(dl, 1), :],
                    osem[q],
                )

            def gather_add(q, j, myrow=myrow):
                return [
                    pltpu.make_async_copy(
                        myrow.at[ibuf.at[q, j, r, 0]],
                        obuf.at[q, j, r, 0],
                        gsem[q],
                    )
                    for r in range(8)
                ]

            for q in range(R):
                idx_fetch(q, q).start()

            def step(g, carry, writeback=writeback, gather_add=gather_add):
                for q in range(R):
                    gi = R * g + q
                    idx_fetch(gi, q).wait()

                    @pl.when(gi >= R)
                    def _():
                        writeback(gi - R, q).wait()

                    for j in range(G):
                        s = gi * G + j
                        vpe = plsc.load_gather(
                            pe_s, [jnp.full((16,), s, jnp.int32)]
                        )
                        for r in range(8):
                            for jj in range(8):
                                obuf[q, j, r, 0, pl.ds(jj * 16, 16)] = vpe
                        for c in gather_add(q, j):
                            c.start(add=True)

                    pq = (q - 1) % R

                    @pl.when(gi >= 1)
                    def _():
                        for j in range(G):
                            for c in gather_add(pq, j):
                                c.wait()
                        writeback(gi - 1, pq).start()

                        @pl.when(gi - 1 + R < ngroups)
                        def _():
                            idx_fetch(gi - 1 + R, pq).start()

                return carry

            lax.fori_loop(0, ngroups // R, step, 0)
            lq = (ngroups - 1) % R
            for j in range(G):
                for c in gather_add(lq, j):
                    c.wait()
            writeback(ngroups - 1, lq).start()
            for q in range(R):
                writeback(ngroups - R + q, q).wait()
            plsc.subcore_barrier()

    return k(tableT, idxT, peT)


@jax.jit
def kernel(input, word_table):
    B, S = input.shape
    D = word_table.shape[1]
    peT = jnp.asarray(np.ascontiguousarray(_PE[:S, :D].T))
    o5 = _sc_embed_planes(word_table.T, input.T.reshape(S, 8, 1, 128), peT)
    return o5.transpose(2, 4, 0, 1, 3).reshape(B, S, D)
